# Initial kernel scaffold; baseline (speedup 1.0000x reference)
import jax
import jax.numpy as jnp
from jax.experimental import pallas as pl


def kernel(x, edge_index, W_enc, b_enc, W1, b1, W2, b2):
    raise NotImplementedError("write your pallas kernel here")


# speedup vs baseline: 3.0525x; 3.0525x over previous
import functools

import jax
import jax.numpy as jnp
from jax import lax
from jax.experimental import pallas as pl
from jax.experimental.pallas import tpu as pltpu
from jax.experimental.pallas import tpu_sc as plsc

N = 10000
D = 128
E = 320000
NL = 5

NC = 2
NS = 16
NW = NC * NS
CHUNK = 128
CHUNKS = 80
EPW = CHUNK * CHUNKS
EPAD = NW * EPW
NPAD = 10112
ZROWS = NPAD // NS
OROWS = 624

ROWBLK = 1000
GRID = N // ROWBLK


def _sc_agg_body(h_hbm, src_hbm, dst_hbm, out_hbm, acc, sidx, didx, rows0, sem0):
    cid = lax.axis_index("c")
    sid = lax.axis_index("s")
    wid = sid * NC + cid
    cbase = wid * CHUNKS

    pltpu.sync_copy(src_hbm.at[pl.ds(cbase, CHUNKS)], sidx)
    pltpu.sync_copy(dst_hbm.at[pl.ds(cbase, CHUNKS)], didx)

    def _zrow(i, c):
        for k in range(D // 16):
            rows0[i, pl.ds(k * 16, 16)] = jnp.zeros((16,), jnp.float32)
        return c
    lax.fori_loop(0, CHUNK, _zrow, 0)
    zbase = sid * ZROWS
    nfull = ZROWS // CHUNK
    for c in range(nfull):
        pltpu.sync_copy(rows0, acc.at[pl.ds(zbase + c * CHUNK, CHUNK)])
    rem = ZROWS - nfull * CHUNK
    if rem:
        pltpu.sync_copy(rows0.at[pl.ds(0, rem)], acc.at[pl.ds(zbase + nfull * CHUNK, rem)])

    plsc.subcore_barrier()

    def _step(j, c):
        pltpu.async_copy(h_hbm.at[sidx.at[j]], rows0, sem0).wait()
        pltpu.sync_copy(rows0, acc.at[didx.at[j]], add=True)
        return c
    lax.fori_loop(0, CHUNKS, _step, 0)

    plsc.subcore_barrier()

    obase = sid * OROWS
    pltpu.sync_copy(acc.at[pl.ds(obase, OROWS)], out_hbm.at[cid, pl.ds(obase, OROWS)])

    @pl.when(sid == NS - 1)
    def _():
        tail = NS * OROWS
        pltpu.sync_copy(acc.at[pl.ds(tail, N - tail)],
                        out_hbm.at[cid, pl.ds(tail, N - tail)])


def _sc_aggregate(h, src2d, dst2d):
    mesh = plsc.VectorSubcoreMesh(
        core_axis_name="c", subcore_axis_name="s", num_cores=NC, num_subcores=NS)
    k = pl.kernel(
        _sc_agg_body,
        out_type=jax.ShapeDtypeStruct((NC, N, D), jnp.float32),
        mesh=mesh,
        scratch_types=[
            pltpu.VMEM_SHARED((NPAD, D), jnp.float32),
            pltpu.VMEM((CHUNKS, CHUNK), jnp.int32),
            pltpu.VMEM((CHUNKS, CHUNK), jnp.int32),
            pltpu.VMEM((CHUNK, D), jnp.float32),
            pltpu.SemaphoreType.DMA,
        ],
    )
    return k(h, src2d, dst2d)


def _enc_body(x_ref, w_ref, b_ref, o_ref):
    z = jnp.dot(x_ref[...], w_ref[...], preferred_element_type=jnp.float32)
    o_ref[...] = jnp.maximum(z + b_ref[...], 0.0)


def _mlp_body(h_ref, p_ref, w1_ref, b1_ref, w2_ref, b2_ref, o_ref):
    z = h_ref[...] + p_ref[0] + p_ref[1]
    a = jnp.maximum(
        jnp.dot(z, w1_ref[...], preferred_element_type=jnp.float32) + b1_ref[...], 0.0)
    o = jnp.dot(a, w2_ref[...], preferred_element_type=jnp.float32) + b2_ref[...]
    o_ref[...] = jnp.maximum(o, 0.0)


def _mlp_final_body(h_ref, p_ref, w1_ref, b1_ref, w2_ref, b2_ref, o_ref, g_ref):
    z = h_ref[...] + p_ref[0] + p_ref[1]
    a = jnp.maximum(
        jnp.dot(z, w1_ref[...], preferred_element_type=jnp.float32) + b1_ref[...], 0.0)
    o = jnp.maximum(
        jnp.dot(a, w2_ref[...], preferred_element_type=jnp.float32) + b2_ref[...], 0.0)
    o_ref[...] = o
    s = jnp.sum(o, axis=0, keepdims=True)
    i = pl.program_id(0)

    @pl.when(i == 0)
    def _():
        g_ref[...] = s

    @pl.when(jnp.logical_and(i > 0, i < GRID - 1))
    def _():
        g_ref[...] = g_ref[...] + s

    @pl.when(i == GRID - 1)
    def _():
        g_ref[...] = (g_ref[...] + s) * jnp.float32(1.0 / N)


_ROW_SPEC = pl.BlockSpec((ROWBLK, D), lambda i: (i, 0))
_P_SPEC = pl.BlockSpec((NC, ROWBLK, D), lambda i: (0, i, 0))
_W_SPEC = pl.BlockSpec((D, D), lambda i: (0, 0))
_B_SPEC = pl.BlockSpec((1, D), lambda i: (0, 0))

_enc_call = pl.pallas_call(
    _enc_body,
    grid=(GRID,),
    in_specs=[_ROW_SPEC, _W_SPEC, _B_SPEC],
    out_specs=_ROW_SPEC,
    out_shape=jax.ShapeDtypeStruct((N, D), jnp.float32),
)

_mlp_call = pl.pallas_call(
    _mlp_body,
    grid=(GRID,),
    in_specs=[_ROW_SPEC, _P_SPEC, _W_SPEC, _B_SPEC, _W_SPEC, _B_SPEC],
    out_specs=_ROW_SPEC,
    out_shape=jax.ShapeDtypeStruct((N, D), jnp.float32),
)

_mlp_final_call = pl.pallas_call(
    _mlp_final_body,
    grid=(GRID,),
    in_specs=[_ROW_SPEC, _P_SPEC, _W_SPEC, _B_SPEC, _W_SPEC, _B_SPEC],
    out_specs=[_ROW_SPEC, pl.BlockSpec((1, D), lambda i: (0, 0))],
    out_shape=[
        jax.ShapeDtypeStruct((N, D), jnp.float32),
        jax.ShapeDtypeStruct((1, D), jnp.float32),
    ],
)


def kernel(x, edge_index, W_enc, b_enc, W1, b1, W2, b2):
    src = edge_index[0]
    dst = edge_index[1]
    pad = EPAD - E
    src2d = jnp.concatenate(
        [src, jnp.zeros((pad,), jnp.int32)]).reshape(NW * CHUNKS, CHUNK)
    dst2d = jnp.concatenate(
        [dst, jnp.full((pad,), N, jnp.int32)]).reshape(NW * CHUNKS, CHUNK)

    h0 = _enc_call(x, W_enc, b_enc.reshape(1, D))
    h = h0
    gsum = None
    for l in range(NL):
        parts = _sc_aggregate(h, src2d, dst2d)
        b1l = b1[l].reshape(1, D)
        b2l = b2[l].reshape(1, D)
        if l < NL - 1:
            h = _mlp_call(h, parts, W1[l], b1l, W2[l], b2l)
        else:
            h, gsum = _mlp_final_call(h, parts, W1[l], b1l, W2[l], b2l)
    return h, gsum.reshape(D), h0


# speedup vs baseline: 3.4137x; 1.1183x over previous
import functools

import jax
import jax.numpy as jnp
from jax import lax
from jax.experimental import pallas as pl
from jax.experimental.pallas import tpu as pltpu
from jax.experimental.pallas import tpu_sc as plsc

N = 10000
D = 128
E = 320000
NL = 5

NC = 2
NS = 16
NW = NC * NS
CHUNK = 128
CHUNKS = 80
EPW = CHUNK * CHUNKS
EPAD = NW * EPW
NPAD = 10112
ZROWS = NPAD // NS
OROWS = 624

ROWBLK = 1000
GRID = N // ROWBLK


def _sc_agg_body(h_hbm, pidx_hbm, out_hbm, acc, pidx, rows0, rows1,
                 gsrc0, gdst0, gsrc1, gdst1, sem0, sem1):
    cid = lax.axis_index("c")
    sid = lax.axis_index("s")
    wid = sid * NC + cid
    cbase = wid * CHUNKS

    pltpu.sync_copy(pidx_hbm.at[pl.ds(cbase, CHUNKS)], pidx)

    def _zrow(i, c):
        for k in range(D // 16):
            rows0[i, pl.ds(k * 16, 16)] = jnp.zeros((16,), jnp.float32)
        return c
    lax.fori_loop(0, CHUNK, _zrow, 0)
    zbase = sid * ZROWS
    nfull = ZROWS // CHUNK
    for c in range(nfull):
        pltpu.sync_copy(rows0, acc.at[pl.ds(zbase + c * CHUNK, CHUNK)])
    rem = ZROWS - nfull * CHUNK
    if rem:
        pltpu.sync_copy(rows0.at[pl.ds(0, rem)], acc.at[pl.ds(zbase + nfull * CHUNK, rem)])

    plsc.subcore_barrier()

    def _unpack(j, gsrc, gdst):
        for k in range(CHUNK // 16):
            v = pidx[j, pl.ds(k * 16, 16)]
            gsrc[pl.ds(k * 16, 16)] = jax.lax.shift_right_logical(v, 14)
            gdst[pl.ds(k * 16, 16)] = jnp.bitwise_and(v, 16383)

    _unpack(0, gsrc0, gdst0)
    pltpu.async_copy(h_hbm.at[gsrc0], rows0, sem0)
    _unpack(1, gsrc1, gdst1)
    pltpu.async_copy(h_hbm.at[gsrc1], rows1, sem1)

    def _step2(g, c):
        j0 = 2 * g
        pltpu.make_async_copy(h_hbm.at[gsrc0], rows0, sem0).wait()
        pltpu.sync_copy(rows0, acc.at[gdst0], add=True)

        @pl.when(j0 + 2 < CHUNKS)
        def _():
            _unpack(j0 + 2, gsrc0, gdst0)
            pltpu.async_copy(h_hbm.at[gsrc0], rows0, sem0)

        pltpu.make_async_copy(h_hbm.at[gsrc1], rows1, sem1).wait()
        pltpu.sync_copy(rows1, acc.at[gdst1], add=True)

        @pl.when(j0 + 3 < CHUNKS)
        def _():
            _unpack(j0 + 3, gsrc1, gdst1)
            pltpu.async_copy(h_hbm.at[gsrc1], rows1, sem1)
        return c
    lax.fori_loop(0, CHUNKS // 2, _step2, 0)

    plsc.subcore_barrier()

    obase = sid * OROWS
    pltpu.sync_copy(acc.at[pl.ds(obase, OROWS)], out_hbm.at[cid, pl.ds(obase, OROWS)])

    @pl.when(sid == NS - 1)
    def _():
        tail = NS * OROWS
        pltpu.sync_copy(acc.at[pl.ds(tail, N - tail)],
                        out_hbm.at[cid, pl.ds(tail, N - tail)])


def _sc_aggregate(h, pidx2d):
    mesh = plsc.VectorSubcoreMesh(
        core_axis_name="c", subcore_axis_name="s", num_cores=NC, num_subcores=NS)
    k = pl.kernel(
        _sc_agg_body,
        out_type=jax.ShapeDtypeStruct((NC, N, D), jnp.float32),
        mesh=mesh,
        scratch_types=[
            pltpu.VMEM_SHARED((NPAD, D), jnp.float32),
            pltpu.VMEM((CHUNKS, CHUNK), jnp.int32),
            pltpu.VMEM((CHUNK, D), jnp.float32),
            pltpu.VMEM((CHUNK, D), jnp.float32),
            pltpu.VMEM((CHUNK,), jnp.int32),
            pltpu.VMEM((CHUNK,), jnp.int32),
            pltpu.VMEM((CHUNK,), jnp.int32),
            pltpu.VMEM((CHUNK,), jnp.int32),
            pltpu.SemaphoreType.DMA,
            pltpu.SemaphoreType.DMA,
        ],
    )
    return k(h, pidx2d)


def _enc_body(x_ref, w_ref, b_ref, o_ref):
    z = jnp.dot(x_ref[...], w_ref[...], preferred_element_type=jnp.float32)
    o_ref[...] = jnp.maximum(z + b_ref[...], 0.0)


def _mlp_body(h_ref, p_ref, w1_ref, b1_ref, w2_ref, b2_ref, o_ref):
    z = h_ref[...] + p_ref[0] + p_ref[1]
    a = jnp.maximum(
        jnp.dot(z, w1_ref[...], preferred_element_type=jnp.float32) + b1_ref[...], 0.0)
    o = jnp.dot(a, w2_ref[...], preferred_element_type=jnp.float32) + b2_ref[...]
    o_ref[...] = jnp.maximum(o, 0.0)


def _mlp_final_body(h_ref, p_ref, w1_ref, b1_ref, w2_ref, b2_ref, o_ref, g_ref):
    z = h_ref[...] + p_ref[0] + p_ref[1]
    a = jnp.maximum(
        jnp.dot(z, w1_ref[...], preferred_element_type=jnp.float32) + b1_ref[...], 0.0)
    o = jnp.maximum(
        jnp.dot(a, w2_ref[...], preferred_element_type=jnp.float32) + b2_ref[...], 0.0)
    o_ref[...] = o
    s = jnp.sum(o, axis=0, keepdims=True)
    i = pl.program_id(0)

    @pl.when(i == 0)
    def _():
        g_ref[...] = s

    @pl.when(jnp.logical_and(i > 0, i < GRID - 1))
    def _():
        g_ref[...] = g_ref[...] + s

    @pl.when(i == GRID - 1)
    def _():
        g_ref[...] = (g_ref[...] + s) * jnp.float32(1.0 / N)


_ROW_SPEC = pl.BlockSpec((ROWBLK, D), lambda i: (i, 0))
_P_SPEC = pl.BlockSpec((NC, ROWBLK, D), lambda i: (0, i, 0))
_W_SPEC = pl.BlockSpec((D, D), lambda i: (0, 0))
_B_SPEC = pl.BlockSpec((1, D), lambda i: (0, 0))

_enc_call = pl.pallas_call(
    _enc_body,
    grid=(GRID,),
    in_specs=[_ROW_SPEC, _W_SPEC, _B_SPEC],
    out_specs=_ROW_SPEC,
    out_shape=jax.ShapeDtypeStruct((N, D), jnp.float32),
)

_mlp_call = pl.pallas_call(
    _mlp_body,
    grid=(GRID,),
    in_specs=[_ROW_SPEC, _P_SPEC, _W_SPEC, _B_SPEC, _W_SPEC, _B_SPEC],
    out_specs=_ROW_SPEC,
    out_shape=jax.ShapeDtypeStruct((N, D), jnp.float32),
)

_mlp_final_call = pl.pallas_call(
    _mlp_final_body,
    grid=(GRID,),
    in_specs=[_ROW_SPEC, _P_SPEC, _W_SPEC, _B_SPEC, _W_SPEC, _B_SPEC],
    out_specs=[_ROW_SPEC, pl.BlockSpec((1, D), lambda i: (0, 0))],
    out_shape=[
        jax.ShapeDtypeStruct((N, D), jnp.float32),
        jax.ShapeDtypeStruct((1, D), jnp.float32),
    ],
)


def kernel(x, edge_index, W_enc, b_enc, W1, b1, W2, b2):
    src = edge_index[0]
    dst = edge_index[1]
    pad = EPAD - E
    packed = src * jnp.int32(16384) + dst
    pidx2d = jnp.concatenate(
        [packed, jnp.full((pad,), N, jnp.int32)]).reshape(NW * CHUNKS, CHUNK)

    h0 = _enc_call(x, W_enc, b_enc.reshape(1, D))
    h = h0
    gsum = None
    for l in range(NL):
        parts = _sc_aggregate(h, pidx2d)
        b1l = b1[l].reshape(1, D)
        b2l = b2[l].reshape(1, D)
        if l < NL - 1:
            h = _mlp_call(h, parts, W1[l], b1l, W2[l], b2l)
        else:
            h, gsum = _mlp_final_call(h, parts, W1[l], b1l, W2[l], b2l)
    return h, gsum.reshape(D), h0


# speedup vs baseline: 3.4906x; 1.0225x over previous
import functools

import jax
import jax.numpy as jnp
from jax import lax
from jax.experimental import pallas as pl
from jax.experimental.pallas import tpu as pltpu
from jax.experimental.pallas import tpu_sc as plsc

N = 10000
D = 128
E = 320000
NL = 5

NC = 2
NS = 16
NW = NC * NS
CHUNK = 64
CHUNKS = 160
PROWS = 80
KBUF = 4
PREF = 3
EPW = CHUNK * CHUNKS
EPAD = NW * EPW
NPAD = 10112
ZROWS = NPAD // NS
OROWS = 624

ROWBLK = 1000
GRID = N // ROWBLK


def _sc_agg_body(h_hbm, pidx_hbm, out_hbm, acc, pidx, rows, gsrc, gdst, sg, ss):
    cid = lax.axis_index("c")
    sid = lax.axis_index("s")
    wid = sid * NC + cid

    pltpu.sync_copy(pidx_hbm.at[pl.ds(wid * PROWS, PROWS)], pidx)

    def _zrow(i, c):
        for k in range(D // 16):
            rows[0][i, pl.ds(k * 16, 16)] = jnp.zeros((16,), jnp.float32)
        return c
    lax.fori_loop(0, CHUNK, _zrow, 0)
    zbase = sid * ZROWS
    nfull = ZROWS // CHUNK
    for c in range(nfull):
        pltpu.sync_copy(rows[0], acc.at[pl.ds(zbase + c * CHUNK, CHUNK)])
    rem = ZROWS - nfull * CHUNK
    if rem:
        pltpu.sync_copy(rows[0].at[pl.ds(0, rem)],
                        acc.at[pl.ds(zbase + nfull * CHUNK, rem)])

    plsc.subcore_barrier()

    def _unpack(j, b):
        row = jax.lax.shift_right_logical(j, 1)
        half = jnp.bitwise_and(j, 1) * CHUNK
        for k in range(CHUNK // 16):
            v = pidx[row, pl.ds(half + k * 16, 16)]
            gsrc[b][pl.ds(k * 16, 16)] = jax.lax.shift_right_logical(v, 14)
            gdst[b][pl.ds(k * 16, 16)] = jnp.bitwise_and(v, 16383)

    def _gather(j, b):
        _unpack(j, b)
        pltpu.async_copy(h_hbm.at[gsrc[b]], rows[b], sg[b])

    for b in range(PREF):
        _gather(jnp.int32(b), b)

    def _visit(j, b):
        pltpu.make_async_copy(h_hbm.at[gsrc[b]], rows[b], sg[b]).wait()
        pltpu.async_copy(rows[b], acc.at[gdst[b]], ss[b], add=True)
        jj = j + PREF
        bb = (b + PREF) % KBUF

        @pl.when(jj < CHUNKS)
        def _():
            @pl.when(jj >= KBUF)
            def _():
                pltpu.make_async_copy(rows[bb], acc.at[gdst[bb]], ss[bb]).wait()
            _gather(jj, bb)

    def _step(g, c):
        j0 = g * KBUF
        for b in range(KBUF):
            _visit(j0 + b, b)
        return c
    lax.fori_loop(0, CHUNKS // KBUF, _step, 0)

    for b in range(KBUF):
        pltpu.make_async_copy(rows[b], acc.at[gdst[b]], ss[b]).wait()

    plsc.subcore_barrier()

    obase = sid * OROWS
    pltpu.sync_copy(acc.at[pl.ds(obase, OROWS)], out_hbm.at[cid, pl.ds(obase, OROWS)])

    @pl.when(sid == NS - 1)
    def _():
        tail = NS * OROWS
        pltpu.sync_copy(acc.at[pl.ds(tail, N - tail)],
                        out_hbm.at[cid, pl.ds(tail, N - tail)])


def _sc_aggregate(h, pidx2d):
    mesh = plsc.VectorSubcoreMesh(
        core_axis_name="c", subcore_axis_name="s", num_cores=NC, num_subcores=NS)
    k = pl.kernel(
        _sc_agg_body,
        out_type=jax.ShapeDtypeStruct((NC, N, D), jnp.float32),
        mesh=mesh,
        scratch_types=[
            pltpu.VMEM_SHARED((NPAD, D), jnp.float32),
            pltpu.VMEM((PROWS, 2 * CHUNK), jnp.int32),
            [pltpu.VMEM((CHUNK, D), jnp.float32) for _ in range(KBUF)],
            [pltpu.VMEM((CHUNK,), jnp.int32) for _ in range(KBUF)],
            [pltpu.VMEM((CHUNK,), jnp.int32) for _ in range(KBUF)],
            [pltpu.SemaphoreType.DMA for _ in range(KBUF)],
            [pltpu.SemaphoreType.DMA for _ in range(KBUF)],
        ],
    )
    return k(h, pidx2d)


def _enc_body(x_ref, w_ref, b_ref, o_ref):
    z = jnp.dot(x_ref[...], w_ref[...], preferred_element_type=jnp.float32)
    o_ref[...] = jnp.maximum(z + b_ref[...], 0.0)


def _mlp_body(h_ref, p_ref, w1_ref, b1_ref, w2_ref, b2_ref, o_ref):
    z = h_ref[...] + p_ref[0] + p_ref[1]
    a = jnp.maximum(
        jnp.dot(z, w1_ref[...], preferred_element_type=jnp.float32) + b1_ref[...], 0.0)
    o = jnp.dot(a, w2_ref[...], preferred_element_type=jnp.float32) + b2_ref[...]
    o_ref[...] = jnp.maximum(o, 0.0)


def _mlp_final_body(h_ref, p_ref, w1_ref, b1_ref, w2_ref, b2_ref, o_ref, g_ref):
    z = h_ref[...] + p_ref[0] + p_ref[1]
    a = jnp.maximum(
        jnp.dot(z, w1_ref[...], preferred_element_type=jnp.float32) + b1_ref[...], 0.0)
    o = jnp.maximum(
        jnp.dot(a, w2_ref[...], preferred_element_type=jnp.float32) + b2_ref[...], 0.0)
    o_ref[...] = o
    s = jnp.sum(o, axis=0, keepdims=True)
    i = pl.program_id(0)

    @pl.when(i == 0)
    def _():
        g_ref[...] = s

    @pl.when(jnp.logical_and(i > 0, i < GRID - 1))
    def _():
        g_ref[...] = g_ref[...] + s

    @pl.when(i == GRID - 1)
    def _():
        g_ref[...] = (g_ref[...] + s) * jnp.float32(1.0 / N)


_ROW_SPEC = pl.BlockSpec((ROWBLK, D), lambda i: (i, 0))
_P_SPEC = pl.BlockSpec((NC, ROWBLK, D), lambda i: (0, i, 0))
_W_SPEC = pl.BlockSpec((D, D), lambda i: (0, 0))
_B_SPEC = pl.BlockSpec((1, D), lambda i: (0, 0))

_enc_call = pl.pallas_call(
    _enc_body,
    grid=(GRID,),
    in_specs=[_ROW_SPEC, _W_SPEC, _B_SPEC],
    out_specs=_ROW_SPEC,
    out_shape=jax.ShapeDtypeStruct((N, D), jnp.float32),
)

_mlp_call = pl.pallas_call(
    _mlp_body,
    grid=(GRID,),
    in_specs=[_ROW_SPEC, _P_SPEC, _W_SPEC, _B_SPEC, _W_SPEC, _B_SPEC],
    out_specs=_ROW_SPEC,
    out_shape=jax.ShapeDtypeStruct((N, D), jnp.float32),
)

_mlp_final_call = pl.pallas_call(
    _mlp_final_body,
    grid=(GRID,),
    in_specs=[_ROW_SPEC, _P_SPEC, _W_SPEC, _B_SPEC, _W_SPEC, _B_SPEC],
    out_specs=[_ROW_SPEC, pl.BlockSpec((1, D), lambda i: (0, 0))],
    out_shape=[
        jax.ShapeDtypeStruct((N, D), jnp.float32),
        jax.ShapeDtypeStruct((1, D), jnp.float32),
    ],
)


def kernel(x, edge_index, W_enc, b_enc, W1, b1, W2, b2):
    src = edge_index[0]
    dst = edge_index[1]
    pad = EPAD - E
    packed = src * jnp.int32(16384) + dst
    pidx2d = jnp.concatenate(
        [packed, jnp.full((pad,), N, jnp.int32)]).reshape(NW * PROWS, 2 * CHUNK)

    h0 = _enc_call(x, W_enc, b_enc.reshape(1, D))
    h = h0
    gsum = None
    for l in range(NL):
        parts = _sc_aggregate(h, pidx2d)
        b1l = b1[l].reshape(1, D)
        b2l = b2[l].reshape(1, D)
        if l < NL - 1:
            h = _mlp_call(h, parts, W1[l], b1l, W2[l], b2l)
        else:
            h, gsum = _mlp_final_call(h, parts, W1[l], b1l, W2[l], b2l)
    return h, gsum.reshape(D), h0
